# baseline (speedup 1.0000x reference)
import functools

import jax
import jax.numpy as jnp
from jax.experimental import pallas as pl
from jax.experimental.pallas import tpu as pltpu

_CANVAS = 1024
_CHUNK_ROWS = 512
_NBUF = 8


def _absdiff_manual(a_hbm, b_hbm, out_ref, a_buf, b_buf, a_sem, b_sem, *,
                    scale, nchunks):
    def start(i, slot):
        rows = pl.ds(i * _CHUNK_ROWS, _CHUNK_ROWS)
        pltpu.make_async_copy(a_hbm.at[rows, :], a_buf.at[slot], a_sem.at[slot]).start()
        pltpu.make_async_copy(b_hbm.at[rows, :], b_buf.at[slot], b_sem.at[slot]).start()

    def wait(i, slot):
        rows = pl.ds(i * _CHUNK_ROWS, _CHUNK_ROWS)
        pltpu.make_async_copy(a_hbm.at[rows, :], a_buf.at[slot], a_sem.at[slot]).wait()
        pltpu.make_async_copy(b_hbm.at[rows, :], b_buf.at[slot], b_sem.at[slot]).wait()

    for s in range(_NBUF):
        start(s, s)

    acc = jnp.zeros((8, 128), dtype=jnp.float32)
    for i in range(nchunks):
        slot = i % _NBUF
        wait(i, slot)
        d = jnp.abs(a_buf[slot] - b_buf[slot])
        acc += jnp.sum(d.reshape(-1, 8, 128), axis=0)
        if i + _NBUF < nchunks:
            start(i + _NBUF, slot)

    out_ref[0, 0] = jnp.sum(acc) * scale


def kernel(sr, hr, patch_cord, h_idx, w_idx):
    b, c, ph, pw = sr.shape
    scale = 1.0 / (b * c * _CANVAS * _CANVAS)
    rows = b * c * ph
    nchunks = rows // _CHUNK_ROWS
    a2 = sr.reshape(rows, pw)
    b2 = hr.reshape(rows, pw)

    out = pl.pallas_call(
        functools.partial(_absdiff_manual, scale=scale, nchunks=nchunks),
        in_specs=[
            pl.BlockSpec(memory_space=pl.ANY),
            pl.BlockSpec(memory_space=pl.ANY),
        ],
        out_specs=pl.BlockSpec(memory_space=pltpu.SMEM),
        out_shape=jax.ShapeDtypeStruct((1, 1), jnp.float32),
        scratch_shapes=[
            pltpu.VMEM((_NBUF, _CHUNK_ROWS, pw), jnp.float32),
            pltpu.VMEM((_NBUF, _CHUNK_ROWS, pw), jnp.float32),
            pltpu.SemaphoreType.DMA((_NBUF,)),
            pltpu.SemaphoreType.DMA((_NBUF,)),
        ],
    )(a2, b2)
    return out[0, 0]


# speedup vs baseline: 1.0353x; 1.0353x over previous
import functools

import jax
import jax.numpy as jnp
from jax.experimental import pallas as pl
from jax.experimental.pallas import tpu as pltpu

_CANVAS = 1024
_CHUNK_ROWS = 2048
_NBUF = 4


def _absdiff_manual(a_hbm, b_hbm, out_ref, a_buf, b_buf, a_sem, b_sem, *,
                    scale, nchunks):
    def start(i, slot):
        rows = pl.ds(i * _CHUNK_ROWS, _CHUNK_ROWS)
        pltpu.make_async_copy(a_hbm.at[rows, :], a_buf.at[slot], a_sem.at[slot]).start()
        pltpu.make_async_copy(b_hbm.at[rows, :], b_buf.at[slot], b_sem.at[slot]).start()

    def wait(i, slot):
        rows = pl.ds(i * _CHUNK_ROWS, _CHUNK_ROWS)
        pltpu.make_async_copy(a_hbm.at[rows, :], a_buf.at[slot], a_sem.at[slot]).wait()
        pltpu.make_async_copy(b_hbm.at[rows, :], b_buf.at[slot], b_sem.at[slot]).wait()

    for s in range(_NBUF):
        start(s, s)

    acc = jnp.zeros((8, 128), dtype=jnp.float32)
    for i in range(nchunks):
        slot = i % _NBUF
        wait(i, slot)
        d = jnp.abs(a_buf[slot] - b_buf[slot])
        acc += jnp.sum(d.reshape(-1, 8, 128), axis=0)
        if i + _NBUF < nchunks:
            start(i + _NBUF, slot)

    out_ref[0, 0] = jnp.sum(acc) * scale


def kernel(sr, hr, patch_cord, h_idx, w_idx):
    b, c, ph, pw = sr.shape
    scale = 1.0 / (b * c * _CANVAS * _CANVAS)
    rows = b * c * ph
    nchunks = rows // _CHUNK_ROWS
    a2 = sr.reshape(rows, pw)
    b2 = hr.reshape(rows, pw)

    out = pl.pallas_call(
        functools.partial(_absdiff_manual, scale=scale, nchunks=nchunks),
        in_specs=[
            pl.BlockSpec(memory_space=pl.ANY),
            pl.BlockSpec(memory_space=pl.ANY),
        ],
        out_specs=pl.BlockSpec(memory_space=pltpu.SMEM),
        out_shape=jax.ShapeDtypeStruct((1, 1), jnp.float32),
        scratch_shapes=[
            pltpu.VMEM((_NBUF, _CHUNK_ROWS, pw), jnp.float32),
            pltpu.VMEM((_NBUF, _CHUNK_ROWS, pw), jnp.float32),
            pltpu.SemaphoreType.DMA((_NBUF,)),
            pltpu.SemaphoreType.DMA((_NBUF,)),
        ],
    )(a2, b2)
    return out[0, 0]
